# Initial kernel scaffold; baseline (speedup 1.0000x reference)
import functools

import jax
import jax.numpy as jnp
from jax import lax
from jax.experimental import pallas as pl
from jax.experimental.pallas import tpu as pltpu, tpu_sc as plsc

N = 10000
D = 128
E = 320000
TD = 128

NC = 2
NS = 16
NW = NC * NS
CW = 128
CH = 79
EPW = CH * CW
EPAD = NW * EPW
NPAD = 10112
RPT = NPAD // NS

_F32 = jnp.float32


def _sc_mesh():
    return plsc.VectorSubcoreMesh(
        core_axis_name="c", subcore_axis_name="s", num_cores=NC, num_subcores=NS
    )


def _deg_body(src_hbm, dst_hbm, outdeg_hbm, indeg_hbm, srcv, dstv, odv, idv):
    c = lax.axis_index("c")
    s = lax.axis_index("s")
    wid = c * NS + s

    z16 = jnp.zeros((16,), _F32)

    def zero_body(i, _):
        odv[pl.ds(i * 16, 16)] = z16
        idv[pl.ds(i * 16, 16)] = z16
        return 0

    lax.fori_loop(0, NPAD // 16, zero_body, 0)

    pltpu.sync_copy(src_hbm.at[wid], srcv)
    pltpu.sync_copy(dst_hbm.at[wid], dstv)

    ones16 = jnp.ones((16,), _F32)

    def acc_body(i, _):
        si = srcv[pl.ds(i * 16, 16)]
        di = dstv[pl.ds(i * 16, 16)]
        plsc.addupdate_scatter(odv, [si], ones16)
        plsc.addupdate_scatter(idv, [di], ones16)
        return 0

    lax.fori_loop(0, EPW // 16, acc_body, 0)

    pltpu.sync_copy(odv, outdeg_hbm.at[wid])
    pltpu.sync_copy(idv, indeg_hbm.at[wid])


_deg_pass = pl.kernel(
    _deg_body,
    out_type=[
        jax.ShapeDtypeStruct((NW, NPAD), _F32),
        jax.ShapeDtypeStruct((NW, NPAD), _F32),
    ],
    mesh=_sc_mesh(),
    scratch_types=[
        pltpu.VMEM((EPW,), jnp.int32),
        pltpu.VMEM((EPW,), jnp.int32),
        pltpu.VMEM((NPAD,), _F32),
        pltpu.VMEM((NPAD,), _F32),
    ],
)


def _agg_body(y_hbm, src_hbm, dst_hbm, out_hbm, srcv, dstv, rows, zbuf, sem, acc):
    c = lax.axis_index("c")
    s = lax.axis_index("s")
    wid = c * NS + s
    base = s * RPT

    z16 = jnp.zeros((16,), _F32)

    def zero_body(i, _):
        r = i // (D // 16)
        j = (i % (D // 16)) * 16
        zbuf[r, pl.ds(j, 16)] = z16
        return 0

    lax.fori_loop(0, (CW * D) // 16, zero_body, 0)

    for k in range(RPT // CW):
        pltpu.sync_copy(zbuf, acc.at[pl.ds(base + k * CW, CW)])
    rem = RPT % CW
    if rem:
        pltpu.sync_copy(
            zbuf.at[pl.ds(0, rem)],
            acc.at[pl.ds(base + (RPT // CW) * CW, rem)],
        )

    pltpu.sync_copy(src_hbm.at[wid], srcv)
    pltpu.sync_copy(dst_hbm.at[wid], dstv)
    plsc.subcore_barrier()

    def chunk_body(g, _):
        pltpu.async_copy(y_hbm.at[srcv.at[g]], rows, sem).wait()
        pltpu.sync_copy(rows, acc.at[dstv.at[g]], add=True)
        return 0

    lax.fori_loop(0, CH, chunk_body, 0)
    plsc.subcore_barrier()

    pltpu.sync_copy(acc.at[pl.ds(base, RPT)], out_hbm.at[c, pl.ds(base, RPT)])


_agg_pass = pl.kernel(
    _agg_body,
    out_type=jax.ShapeDtypeStruct((NC, NPAD, D), _F32),
    mesh=_sc_mesh(),
    scratch_types=[
        pltpu.VMEM((CH, CW), jnp.int32),
        pltpu.VMEM((CH, CW), jnp.int32),
        pltpu.VMEM((CW, D), _F32),
        pltpu.VMEM((CW, D), _F32),
        pltpu.SemaphoreType.DMA,
        pltpu.VMEM_SHARED((NPAD, D), _F32),
    ],
)


_BLK = 1264


def _norm_body(od_ref, id_ref, x_ref, y_ref, ns_ref, nd_ref):
    ones = jnp.ones((NW, 1), _F32)
    od = lax.dot_general(od_ref[...], ones, (((0,), (0,)), ((), ())),
                         preferred_element_type=_F32)
    idg = lax.dot_general(id_ref[...], ones, (((0,), (0,)), ((), ())),
                          preferred_element_type=_F32)
    ns = lax.rsqrt(jnp.maximum(od, 1.0))
    nd = lax.rsqrt(jnp.maximum(idg, 1.0))
    ns_ref[...] = ns
    nd_ref[...] = nd
    y_ref[...] = x_ref[...] * ns


def _norm_pass(od, idp, xpad):
    return pl.pallas_call(
        _norm_body,
        grid=(NPAD // _BLK,),
        in_specs=[
            pl.BlockSpec((NW, _BLK), lambda i: (0, i)),
            pl.BlockSpec((NW, _BLK), lambda i: (0, i)),
            pl.BlockSpec((_BLK, D), lambda i: (i, 0)),
        ],
        out_specs=[
            pl.BlockSpec((_BLK, D), lambda i: (i, 0)),
            pl.BlockSpec((_BLK, 1), lambda i: (i, 0)),
            pl.BlockSpec((_BLK, 1), lambda i: (i, 0)),
        ],
        out_shape=[
            jax.ShapeDtypeStruct((NPAD, D), _F32),
            jax.ShapeDtypeStruct((NPAD, 1), _F32),
            jax.ShapeDtypeStruct((NPAD, 1), _F32),
        ],
    )(od, idp, xpad)


def _mid_body(p_ref, nd_ref, ns_ref, w_ref, b_ref, y_ref):
    agg = (p_ref[0] + p_ref[1]) * nd_ref[...]
    r = lax.dot_general(agg, w_ref[...], (((1,), (0,)), ((), ())),
                        preferred_element_type=_F32,
                        precision=lax.Precision.HIGHEST)
    r = jnp.maximum(r + b_ref[...], 0.0)
    y_ref[...] = r * ns_ref[...]


def _mid_pass(parts, nd, ns, W0, b0):
    return pl.pallas_call(
        _mid_body,
        grid=(NPAD // _BLK,),
        in_specs=[
            pl.BlockSpec((NC, _BLK, D), lambda i: (0, i, 0)),
            pl.BlockSpec((_BLK, 1), lambda i: (i, 0)),
            pl.BlockSpec((_BLK, 1), lambda i: (i, 0)),
            pl.BlockSpec((D, D), lambda i: (0, 0)),
            pl.BlockSpec((1, D), lambda i: (0, 0)),
        ],
        out_specs=pl.BlockSpec((_BLK, D), lambda i: (i, 0)),
        out_shape=jax.ShapeDtypeStruct((NPAD, D), _F32),
    )(parts, nd, ns, W0, b0.reshape(1, D))


def _fin_body(p_ref, nd_ref, w1_ref, b1_ref, t_ref, tw_ref, tb_ref,
              wih_ref, bih_ref, whh_ref, bhh_ref, out_ref):
    agg = (p_ref[0] + p_ref[1]) * nd_ref[...]
    h = lax.dot_general(agg, w1_ref[...], (((1,), (0,)), ((), ())),
                        preferred_element_type=_F32,
                        precision=lax.Precision.HIGHEST)
    h = jnp.maximum(h + b1_ref[...], 0.0)

    te = jnp.cos(t_ref[0, 0] * tw_ref[...] + tb_ref[...])
    wih = wih_ref[...]
    gi_h = lax.dot_general(h, wih[:, :D], (((1,), (1,)), ((), ())),
                           preferred_element_type=_F32,
                           precision=lax.Precision.HIGHEST)
    gi_te = lax.dot_general(te, wih[:, D:], (((1,), (1,)), ((), ())),
                            preferred_element_type=_F32,
                            precision=lax.Precision.HIGHEST)
    gi = gi_h + gi_te + bih_ref[...]
    gh = lax.dot_general(h, whh_ref[...], (((1,), (1,)), ((), ())),
                         preferred_element_type=_F32,
                         precision=lax.Precision.HIGHEST) + bhh_ref[...]

    i_r, i_z, i_n = gi[:, :D], gi[:, D:2 * D], gi[:, 2 * D:]
    h_r, h_z, h_n = gh[:, :D], gh[:, D:2 * D], gh[:, 2 * D:]
    r = jax.nn.sigmoid(i_r + h_r)
    z = jax.nn.sigmoid(i_z + h_z)
    n = jnp.tanh(i_n + r * h_n)
    out_ref[...] = (1.0 - z) * n + z * h


def _fin_pass(parts, nd, W1, b1, t, time_w, time_b, W_ih, b_ih, W_hh, b_hh):
    return pl.pallas_call(
        _fin_body,
        grid=(NPAD // _BLK,),
        in_specs=[
            pl.BlockSpec((NC, _BLK, D), lambda i: (0, i, 0)),
            pl.BlockSpec((_BLK, 1), lambda i: (i, 0)),
            pl.BlockSpec((D, D), lambda i: (0, 0)),
            pl.BlockSpec((1, D), lambda i: (0, 0)),
            pl.BlockSpec((1, 1), lambda i: (0, 0)),
            pl.BlockSpec((1, TD), lambda i: (0, 0)),
            pl.BlockSpec((1, TD), lambda i: (0, 0)),
            pl.BlockSpec((3 * D, D + TD), lambda i: (0, 0)),
            pl.BlockSpec((1, 3 * D), lambda i: (0, 0)),
            pl.BlockSpec((3 * D, D), lambda i: (0, 0)),
            pl.BlockSpec((1, 3 * D), lambda i: (0, 0)),
        ],
        out_specs=pl.BlockSpec((_BLK, D), lambda i: (i, 0)),
        out_shape=jax.ShapeDtypeStruct((NPAD, D), _F32),
    )(parts, nd, W1, b1.reshape(1, D), t.reshape(1, 1),
      time_w.reshape(1, TD), time_b.reshape(1, TD),
      W_ih, b_ih.reshape(1, 3 * D), W_hh, b_hh.reshape(1, 3 * D))


def kernel(hidden, edge_index, t, time_w, time_b, W0, b0, W1, b1,
           W_ih, b_ih, W_hh, b_hh):
    src = edge_index[0]
    dst = edge_index[1]
    pad = jnp.full((EPAD - E,), N, jnp.int32)
    src_p = jnp.concatenate([src, pad])
    dst_p = jnp.concatenate([dst, pad])
    src2 = src_p.reshape(NW, EPW)
    dst2 = dst_p.reshape(NW, EPW)
    src3 = src_p.reshape(NW, CH, CW)
    dst3 = dst_p.reshape(NW, CH, CW)
    xpad = jnp.zeros((NPAD, D), _F32).at[:N].set(hidden)

    od_parts, id_parts = _deg_pass(src2, dst2)
    y0, ns, nd = _norm_pass(od_parts, id_parts, xpad)
    parts1 = _agg_pass(y0, src3, dst3)
    y1 = _mid_pass(parts1, nd, ns, W0, b0)
    parts2 = _agg_pass(y1, src3, dst3)
    out = _fin_pass(parts2, nd, W1, b1, t, time_w, time_b,
                    W_ih, b_ih, W_hh, b_hh)
    return out[:N]


# speedup vs baseline: 3.8383x; 3.8383x over previous
import functools

import jax
import jax.numpy as jnp
from jax import lax
from jax.experimental import pallas as pl
from jax.experimental.pallas import tpu as pltpu, tpu_sc as plsc

N = 10000
D = 128
E = 320000
TD = 128

NC = 2
NS = 16
NW = NC * NS
CW = 128
CH = 79
EPW = CH * CW
EPAD = NW * EPW
NPAD = 10240
RPT = NPAD // NS

_F32 = jnp.float32


def _sc_mesh():
    return plsc.VectorSubcoreMesh(
        core_axis_name="c", subcore_axis_name="s", num_cores=NC, num_subcores=NS
    )


def _deg_body(src_hbm, dst_hbm, outdeg_hbm, indeg_hbm, srcv, dstv, odv, idv):
    c = lax.axis_index("c")
    s = lax.axis_index("s")
    wid = c * NS + s

    z16 = jnp.zeros((16,), _F32)

    def zero_body(i, _):
        odv[pl.ds(i * 16, 16)] = z16
        idv[pl.ds(i * 16, 16)] = z16
        return 0

    lax.fori_loop(0, NPAD // 16, zero_body, 0)

    pltpu.sync_copy(src_hbm.at[wid], srcv)
    pltpu.sync_copy(dst_hbm.at[wid], dstv)

    ones16 = jnp.ones((16,), _F32)

    def acc_body(i, _):
        si = srcv[pl.ds(i * 16, 16)]
        di = dstv[pl.ds(i * 16, 16)]
        plsc.addupdate_scatter(odv, [si], ones16)
        plsc.addupdate_scatter(idv, [di], ones16)
        return 0

    lax.fori_loop(0, EPW // 16, acc_body, 0)

    pltpu.sync_copy(odv, outdeg_hbm.at[wid])
    pltpu.sync_copy(idv, indeg_hbm.at[wid])


_deg_pass = pl.kernel(
    _deg_body,
    out_type=[
        jax.ShapeDtypeStruct((NW, NPAD), _F32),
        jax.ShapeDtypeStruct((NW, NPAD), _F32),
    ],
    mesh=_sc_mesh(),
    compiler_params=pltpu.CompilerParams(needs_layout_passes=False),
    scratch_types=[
        pltpu.VMEM((EPW,), jnp.int32),
        pltpu.VMEM((EPW,), jnp.int32),
        pltpu.VMEM((NPAD,), _F32),
        pltpu.VMEM((NPAD,), _F32),
    ],
)


def _agg_body(y_hbm, src_hbm, dst_hbm, out_hbm, srcv, dstv, rows, sem, acc):
    c = lax.axis_index("c")
    s = lax.axis_index("s")
    wid = c * NS + s
    base = s * RPT

    z16 = jnp.zeros((16,), _F32)

    def zero_body(i, _):
        r = i // (D // 16)
        j = (i % (D // 16)) * 16
        rows[r, pl.ds(j, 16)] = z16
        return 0

    lax.fori_loop(0, (CW * D) // 16, zero_body, 0)

    for k in range(RPT // CW):
        pltpu.sync_copy(rows, acc.at[pl.ds(base + k * CW, CW)])

    pltpu.sync_copy(src_hbm.at[wid], srcv)
    pltpu.sync_copy(dst_hbm.at[wid], dstv)
    plsc.subcore_barrier()

    def chunk_body(g, _):
        pltpu.async_copy(y_hbm.at[srcv.at[g]], rows, sem).wait()
        pltpu.sync_copy(rows, acc.at[dstv.at[g]], add=True)
        return 0

    lax.fori_loop(0, CH, chunk_body, 0)
    plsc.subcore_barrier()

    pltpu.sync_copy(acc.at[pl.ds(base, RPT)], out_hbm.at[c, pl.ds(base, RPT)])


_agg_pass = pl.kernel(
    _agg_body,
    out_type=jax.ShapeDtypeStruct((NC, NPAD, D), _F32),
    mesh=_sc_mesh(),
    compiler_params=pltpu.CompilerParams(needs_layout_passes=False),
    scratch_types=[
        pltpu.VMEM((CH, CW), jnp.int32),
        pltpu.VMEM((CH, CW), jnp.int32),
        pltpu.VMEM((CW, D), _F32),
        pltpu.SemaphoreType.DMA,
        pltpu.VMEM_SHARED((NPAD, D), _F32),
    ],
)


_BLK = 1280


def _norm_body(od_ref, id_ref, x_ref, y_ref, ns_ref, nd_ref):
    ones = jnp.ones((NW, 1), _F32)
    od = lax.dot_general(od_ref[...], ones, (((0,), (0,)), ((), ())),
                         preferred_element_type=_F32)
    idg = lax.dot_general(id_ref[...], ones, (((0,), (0,)), ((), ())),
                          preferred_element_type=_F32)
    ns = lax.rsqrt(jnp.maximum(od, 1.0))
    nd = lax.rsqrt(jnp.maximum(idg, 1.0))
    ns_ref[...] = ns
    nd_ref[...] = nd
    y_ref[...] = x_ref[...] * ns


def _norm_pass(od, idp, xpad):
    return pl.pallas_call(
        _norm_body,
        grid=(NPAD // _BLK,),
        in_specs=[
            pl.BlockSpec((NW, _BLK), lambda i: (0, i)),
            pl.BlockSpec((NW, _BLK), lambda i: (0, i)),
            pl.BlockSpec((_BLK, D), lambda i: (i, 0)),
        ],
        out_specs=[
            pl.BlockSpec((_BLK, D), lambda i: (i, 0)),
            pl.BlockSpec((_BLK, 1), lambda i: (i, 0)),
            pl.BlockSpec((_BLK, 1), lambda i: (i, 0)),
        ],
        out_shape=[
            jax.ShapeDtypeStruct((NPAD, D), _F32),
            jax.ShapeDtypeStruct((NPAD, 1), _F32),
            jax.ShapeDtypeStruct((NPAD, 1), _F32),
        ],
    )(od, idp, xpad)


def _mid_body(p_ref, nd_ref, ns_ref, w_ref, b_ref, y_ref):
    agg = (p_ref[0] + p_ref[1]) * nd_ref[...]
    r = lax.dot_general(agg, w_ref[...], (((1,), (0,)), ((), ())),
                        preferred_element_type=_F32,
                        precision=lax.Precision.HIGHEST)
    r = jnp.maximum(r + b_ref[...], 0.0)
    y_ref[...] = r * ns_ref[...]


def _mid_pass(parts, nd, ns, W0, b0):
    return pl.pallas_call(
        _mid_body,
        grid=(NPAD // _BLK,),
        in_specs=[
            pl.BlockSpec((NC, _BLK, D), lambda i: (0, i, 0)),
            pl.BlockSpec((_BLK, 1), lambda i: (i, 0)),
            pl.BlockSpec((_BLK, 1), lambda i: (i, 0)),
            pl.BlockSpec((D, D), lambda i: (0, 0)),
            pl.BlockSpec((1, D), lambda i: (0, 0)),
        ],
        out_specs=pl.BlockSpec((_BLK, D), lambda i: (i, 0)),
        out_shape=jax.ShapeDtypeStruct((NPAD, D), _F32),
    )(parts, nd, ns, W0, b0.reshape(1, D))


def _fin_body(p_ref, nd_ref, w1_ref, b1_ref, t_ref, tw_ref, tb_ref,
              wih_ref, bih_ref, whh_ref, bhh_ref, out_ref):
    agg = (p_ref[0] + p_ref[1]) * nd_ref[...]
    h = lax.dot_general(agg, w1_ref[...], (((1,), (0,)), ((), ())),
                        preferred_element_type=_F32,
                        precision=lax.Precision.HIGHEST)
    h = jnp.maximum(h + b1_ref[...], 0.0)

    te = jnp.cos(t_ref[0, 0] * tw_ref[...] + tb_ref[...])
    wih = wih_ref[...]
    gi_h = lax.dot_general(h, wih[:, :D], (((1,), (1,)), ((), ())),
                           preferred_element_type=_F32,
                           precision=lax.Precision.HIGHEST)
    gi_te = lax.dot_general(te, wih[:, D:], (((1,), (1,)), ((), ())),
                            preferred_element_type=_F32,
                            precision=lax.Precision.HIGHEST)
    gi = gi_h + gi_te + bih_ref[...]
    gh = lax.dot_general(h, whh_ref[...], (((1,), (1,)), ((), ())),
                         preferred_element_type=_F32,
                         precision=lax.Precision.HIGHEST) + bhh_ref[...]

    i_r, i_z, i_n = gi[:, :D], gi[:, D:2 * D], gi[:, 2 * D:]
    h_r, h_z, h_n = gh[:, :D], gh[:, D:2 * D], gh[:, 2 * D:]
    r = jax.nn.sigmoid(i_r + h_r)
    z = jax.nn.sigmoid(i_z + h_z)
    n = jnp.tanh(i_n + r * h_n)
    out_ref[...] = (1.0 - z) * n + z * h


def _fin_pass(parts, nd, W1, b1, t, time_w, time_b, W_ih, b_ih, W_hh, b_hh):
    return pl.pallas_call(
        _fin_body,
        grid=(NPAD // _BLK,),
        in_specs=[
            pl.BlockSpec((NC, _BLK, D), lambda i: (0, i, 0)),
            pl.BlockSpec((_BLK, 1), lambda i: (i, 0)),
            pl.BlockSpec((D, D), lambda i: (0, 0)),
            pl.BlockSpec((1, D), lambda i: (0, 0)),
            pl.BlockSpec((1, 1), lambda i: (0, 0)),
            pl.BlockSpec((1, TD), lambda i: (0, 0)),
            pl.BlockSpec((1, TD), lambda i: (0, 0)),
            pl.BlockSpec((3 * D, D + TD), lambda i: (0, 0)),
            pl.BlockSpec((1, 3 * D), lambda i: (0, 0)),
            pl.BlockSpec((3 * D, D), lambda i: (0, 0)),
            pl.BlockSpec((1, 3 * D), lambda i: (0, 0)),
        ],
        out_specs=pl.BlockSpec((_BLK, D), lambda i: (i, 0)),
        out_shape=jax.ShapeDtypeStruct((NPAD, D), _F32),
    )(parts, nd, W1, b1.reshape(1, D), t.reshape(1, 1),
      time_w.reshape(1, TD), time_b.reshape(1, TD),
      W_ih, b_ih.reshape(1, 3 * D), W_hh, b_hh.reshape(1, 3 * D))


def kernel(hidden, edge_index, t, time_w, time_b, W0, b0, W1, b1,
           W_ih, b_ih, W_hh, b_hh):
    src = edge_index[0]
    dst = edge_index[1]
    pad = jnp.full((EPAD - E,), N, jnp.int32)
    src_p = jnp.concatenate([src, pad])
    dst_p = jnp.concatenate([dst, pad])
    src2 = src_p.reshape(NW, EPW)
    dst2 = dst_p.reshape(NW, EPW)
    src3 = src_p.reshape(NW, CH, CW)
    dst3 = dst_p.reshape(NW, CH, CW)
    xpad = jnp.zeros((NPAD, D), _F32).at[:N].set(hidden)

    od_parts, id_parts = _deg_pass(src2, dst2)
    y0, ns, nd = _norm_pass(od_parts, id_parts, xpad)
    parts1 = _agg_pass(y0, src3, dst3)
    y1 = _mid_pass(parts1, nd, ns, W0, b0)
    parts2 = _agg_pass(y1, src3, dst3)
    out = _fin_pass(parts2, nd, W1, b1, t, time_w, time_b,
                    W_ih, b_ih, W_hh, b_hh)
    return out[:N]
